# baseline (device time: 227716 ns/iter reference)
import jax
import jax.numpy as jnp
from jax import lax
from jax.experimental import pallas as pl
from jax.experimental.pallas import tpu as pltpu

N_DEV = 4
HQ = 8
DH = 128
SQ = 1024
SKV = 1024
D_MODEL = 1024
SCALE = 0.08838834764831843


def _attend(q_all, k_all, v_all, ctx_ref):
    for h in range(HQ):
        cs = slice(h * DH, (h + 1) * DH)
        q4 = q_all[:, cs].reshape(4, 256, DH)
        k4 = k_all[:, cs].reshape(4, 256, DH)
        v4 = v_all[:, cs].reshape(4, 256, DH)
        scores = lax.dot_general(
            q4, k4, (((2,), (2,)), ((0,), (0,))),
            preferred_element_type=jnp.float32) * SCALE
        m = jnp.max(scores, axis=2, keepdims=True)
        e = jnp.exp(scores - m)
        z = jnp.sum(e, axis=2, keepdims=True)
        w = (e / z).astype(jnp.bfloat16)
        c = lax.dot_general(
            w, v4, (((2,), (1,)), ((0,), (0,))),
            preferred_element_type=jnp.float32)
        ctx_ref[:, cs] = c.astype(jnp.bfloat16).reshape(SQ, DH)


def _body(x_ref, wq_ref, wo_ref, k_ref, v_ref, out_ref,
          wq_comm, wo_comm, ctx_ref,
          wq_ssem, wq_rsem, wo_ssem, wo_rsem):
    my = lax.axis_index("i")
    left = lax.rem(my + N_DEV - 1, N_DEV)
    right = lax.rem(my + 1, N_DEV)

    barrier = pltpu.get_barrier_semaphore()
    pl.semaphore_signal(barrier, inc=1, device_id=(left,),
                        device_id_type=pl.DeviceIdType.MESH)
    pl.semaphore_signal(barrier, inc=1, device_id=(right,),
                        device_id_type=pl.DeviceIdType.MESH)
    pl.semaphore_wait(barrier, 2)

    wq_comm[0] = wq_ref[...]
    wo_comm[0] = wo_ref[...]
    out_ref[0] = jnp.zeros((SQ, D_MODEL), jnp.float32)

    x = x_ref[...]

    for s in range(N_DEV):
        if s < N_DEV - 1:
            wq_rdma = pltpu.make_async_remote_copy(
                src_ref=wq_comm.at[s], dst_ref=wq_comm.at[s + 1],
                send_sem=wq_ssem.at[s], recv_sem=wq_rsem.at[s],
                device_id=(right,), device_id_type=pl.DeviceIdType.MESH)
            wo_rdma = pltpu.make_async_remote_copy(
                src_ref=wo_comm.at[s], dst_ref=wo_comm.at[s + 1],
                send_sem=wo_ssem.at[s], recv_sem=wo_rsem.at[s],
                device_id=(right,), device_id_type=pl.DeviceIdType.MESH)
            wq_rdma.start()
            wo_rdma.start()

        g = lax.rem(my + N_DEV - s, N_DEV)

        q_all = jnp.dot(x, wq_comm[s],
                        preferred_element_type=jnp.float32).astype(jnp.bfloat16)
        _attend(q_all, k_ref[g], v_ref[g], ctx_ref)
        out_ref[0] += jnp.dot(ctx_ref[...], wo_comm[s],
                              preferred_element_type=jnp.float32)

        if s < N_DEV - 1:
            wq_rdma.wait()
            wo_rdma.wait()


def _permute_rows(a):
    return a.reshape(4, 4, 64, *a.shape[1:]).swapaxes(0, 1).reshape(a.shape)


def kernel(x, Wq, K_ext, V_ext, Wo):
    my = lax.axis_index("i")

    xb = _permute_rows(x[0].astype(jnp.bfloat16))
    wq = Wq.astype(jnp.bfloat16)
    wo = Wo.astype(jnp.bfloat16)

    kb = lax.dynamic_index_in_dim(K_ext, my, 0, keepdims=False)
    vb = lax.dynamic_index_in_dim(V_ext, my, 0, keepdims=False)
    kb = _permute_rows(kb.astype(jnp.bfloat16).reshape(SKV, 4 * HQ * DH))
    vb = _permute_rows(vb.astype(jnp.bfloat16).reshape(SKV, 4 * HQ * DH))
    kb = kb.reshape(SKV, N_DEV, HQ * DH).transpose(1, 0, 2)
    vb = vb.reshape(SKV, N_DEV, HQ * DH).transpose(1, 0, 2)

    out = pl.pallas_call(
        _body,
        out_shape=jax.ShapeDtypeStruct((1, SQ, D_MODEL), jnp.float32),
        in_specs=[pl.BlockSpec(memory_space=pltpu.VMEM)] * 5,
        out_specs=pl.BlockSpec(memory_space=pltpu.VMEM),
        scratch_shapes=[
            pltpu.VMEM((N_DEV, D_MODEL, HQ * DH), jnp.bfloat16),
            pltpu.VMEM((N_DEV, HQ * DH, D_MODEL), jnp.bfloat16),
            pltpu.VMEM((SQ, HQ * DH), jnp.bfloat16),
            pltpu.SemaphoreType.DMA((N_DEV - 1,)),
            pltpu.SemaphoreType.DMA((N_DEV - 1,)),
            pltpu.SemaphoreType.DMA((N_DEV - 1,)),
            pltpu.SemaphoreType.DMA((N_DEV - 1,)),
        ],
        compiler_params=pltpu.CompilerParams(collective_id=0),
    )(xb, wq, wo, kb, vb)
    return _permute_rows(out[0])[None]
